# baseline (speedup 1.0000x reference)
import functools

import jax
import jax.numpy as jnp
from jax import lax
from jax.experimental import pallas as pl
from jax.experimental.pallas import tpu as pltpu
from jax.experimental.pallas import tpu_sc as plsc

_V = 1000000
_D = 32
_OUT = 10
_PD = 16
_B = 16384
_H = 200

_NC = 2
_NS = 16
_NW = _NC * _NS
_SPW = _B // _NW
_XW = 256
_C = 4
_NCHUNK = _SPW // _C
_LA = 104
_LB = 96
_L = 16
_UNROLL = 8

_CB = 8192
_W = 1000
_NCK = _V // _W


def _project_body(w_ref, tT_ref, o_ref):
    w = w_ref[...] * (1.0 / _H)
    blk = tT_ref[...]
    o_ref[...] = lax.dot_general(
        w, blk,
        dimension_numbers=(((1,), (0,)), ((), ())),
        preferred_element_type=jnp.float32,
    )


_project = pl.pallas_call(
    _project_body,
    grid=((_V + _CB - 1) // _CB,),
    in_specs=[
        pl.BlockSpec((_PD, _D), lambda i: (0, 0)),
        pl.BlockSpec((_D, _CB), lambda i: (0, i)),
    ],
    out_specs=pl.BlockSpec((_PD, _CB), lambda i: (0, i)),
    out_shape=jax.ShapeDtypeStruct((_PD, _V), jnp.float32),
)


def _pack_body(tPT_hbm, tP_hbm, in_v, out_v, sem):
    wid = lax.axis_index("s") * _NC + lax.axis_index("c")
    lane_base = lax.iota(jnp.int32, _L) * _W

    def chunk(k, carry):
        cid = wid + _NW * k

        @pl.when(cid < _NCK)
        def _():
            c0 = cid * _W
            for d in range(_PD):
                pltpu.async_copy(
                    tPT_hbm.at[d, pl.ds(c0, _W)],
                    in_v.at[pl.ds(d * _W, _W)], sem)
            for d in range(_PD):
                pltpu.make_async_copy(
                    tPT_hbm.at[d, pl.ds(c0, _W)],
                    in_v.at[pl.ds(d * _W, _W)], sem).wait()

            def row(i, c2):
                v = plsc.load_gather(in_v, [lane_base + i])
                out_v[i, :] = v
                return c2

            lax.fori_loop(0, _W, row, 0)
            pltpu.sync_copy(out_v, tP_hbm.at[pl.ds(c0, _W), :])

        return carry

    lax.fori_loop(0, (_NCK + _NW - 1) // _NW, chunk, 0)


_pack = functools.partial(
    pl.kernel,
    out_type=jax.ShapeDtypeStruct((_V, _PD), jnp.float32),
    mesh=plsc.VectorSubcoreMesh(core_axis_name="c", subcore_axis_name="s"),
    compiler_params=pltpu.CompilerParams(
        use_tc_tiling_on_sc=False, needs_layout_passes=False),
    scratch_types=[
        pltpu.VMEM((_PD * _W,), jnp.float32),
        pltpu.VMEM((_W, _PD), jnp.float32),
        pltpu.SemaphoreType.DMA,
    ],
)(_pack_body)


def _pool_body(x_hbm, table_hbm, out_hbm, idx_v, rowsA, rowsB, out_v,
               sem0, sem1):
    wid = lax.axis_index("s") * _NC + lax.axis_index("c")
    base = wid * _SPW
    sems = (sem0, sem1)

    def fire(ci, b):
        r0 = base + ci * _C
        pltpu.sync_copy(x_hbm.at[pl.ds(r0, _C), :], idx_v.at[b])
        for s in range(_C):
            pltpu.async_copy(
                table_hbm.at[idx_v.at[b, s, pl.ds(0, _LA)]],
                rowsA.at[b, s], sems[b])
            pltpu.async_copy(
                table_hbm.at[idx_v.at[b, s, pl.ds(_LA, _LB)]],
                rowsB.at[b, s], sems[b])

    def drain(b):
        for s in range(_C):
            pltpu.make_async_copy(
                table_hbm.at[idx_v.at[b, s, pl.ds(0, _LA)]],
                rowsA.at[b, s], sems[b]).wait()
            pltpu.make_async_copy(
                table_hbm.at[idx_v.at[b, s, pl.ds(_LA, _LB)]],
                rowsB.at[b, s], sems[b]).wait()

    def accumulate(ci, b):
        for s in range(_C):
            def bodyA(k, a):
                for u in range(_UNROLL):
                    a = a + rowsA[b, s, k * _UNROLL + u, :]
                return a

            def bodyB(k, a):
                for u in range(_UNROLL):
                    a = a + rowsB[b, s, k * _UNROLL + u, :]
                return a

            z = jnp.zeros((_L,), jnp.float32)
            a = lax.fori_loop(0, _LA // _UNROLL, bodyA, z)
            a = lax.fori_loop(0, _LB // _UNROLL, bodyB, a)
            out_v[ci * _C + s, :] = a

    fire(0, 0)

    def body(i, carry):
        cc = 2 * i
        fire(cc + 1, 1)
        drain(0)
        accumulate(cc, 0)

        @pl.when(i + 1 < _NCHUNK // 2)
        def _():
            fire(cc + 2, 0)

        drain(1)
        accumulate(cc + 1, 1)
        return carry

    lax.fori_loop(0, _NCHUNK // 2, body, 0)
    pltpu.sync_copy(out_v, out_hbm.at[pl.ds(base, _SPW), :])


_pool = functools.partial(
    pl.kernel,
    out_type=jax.ShapeDtypeStruct((_B, _PD), jnp.float32),
    mesh=plsc.VectorSubcoreMesh(core_axis_name="c", subcore_axis_name="s"),
    compiler_params=pltpu.CompilerParams(
        use_tc_tiling_on_sc=False, needs_layout_passes=False),
    scratch_types=[
        pltpu.VMEM((2, _C, _XW), jnp.int32),
        pltpu.VMEM((2, _C, _LA, _PD), jnp.float32),
        pltpu.VMEM((2, _C, _LB, _PD), jnp.float32),
        pltpu.VMEM((_SPW, _PD), jnp.float32),
        pltpu.SemaphoreType.DMA,
        pltpu.SemaphoreType.DMA,
    ],
)(_pool_body)


def _take_body(p_ref, o_ref):
    o_ref[...] = p_ref[:, : _OUT]


_take = pl.pallas_call(
    _take_body,
    out_shape=jax.ShapeDtypeStruct((_B, _OUT), jnp.float32),
)


def kernel(x, emb_table, fc_w):
    x_p = jnp.pad(x, ((0, 0), (0, _XW - _H)))
    w16 = jnp.pad(fc_w, ((0, _PD - _OUT), (0, 0)))
    tPT = _project(w16, emb_table.T)
    tP = _pack(tPT)
    pooled = _pool(x_p, tP)
    return _take(pooled)


# speedup vs baseline: 2.4526x; 2.4526x over previous
import functools

import jax
import jax.numpy as jnp
from jax import lax
from jax.experimental import pallas as pl
from jax.experimental.pallas import tpu as pltpu
from jax.experimental.pallas import tpu_sc as plsc

_D = 32
_OUT = 10
_B = 16384
_H = 200

_NC = 2
_NS = 16
_NW = _NC * _NS
_SPW = _B // _NW
_C = 4
_NCHUNK = _SPW // _C
_LA = 104
_LB = 96
_L = 16
_UNROLL = 8


def _pool_body(x_hbm, table_hbm, out_hbm, idx_v, rowsA, rowsB, out_v,
               sem0, sem1):
    wid = lax.axis_index("s") * _NC + lax.axis_index("c")
    base = wid * _SPW
    sems = (sem0, sem1)

    def fire(ci, b):
        r0 = base + ci * _C
        pltpu.sync_copy(x_hbm.at[pl.ds(r0, _C), :], idx_v.at[b])
        for s in range(_C):
            pltpu.async_copy(
                table_hbm.at[idx_v.at[b, s, pl.ds(0, _LA)]],
                rowsA.at[b, s], sems[b])
            pltpu.async_copy(
                table_hbm.at[idx_v.at[b, s, pl.ds(_LA, _LB)]],
                rowsB.at[b, s], sems[b])

    def drain(b):
        for s in range(_C):
            pltpu.make_async_copy(
                table_hbm.at[idx_v.at[b, s, pl.ds(0, _LA)]],
                rowsA.at[b, s], sems[b]).wait()
            pltpu.make_async_copy(
                table_hbm.at[idx_v.at[b, s, pl.ds(_LA, _LB)]],
                rowsB.at[b, s], sems[b]).wait()

    def accumulate(ci, b):
        for s in range(_C):
            def bodyA(k, acc):
                a0, a1 = acc
                for u in range(_UNROLL):
                    l = k * _UNROLL + u
                    a0 = a0 + rowsA[b, s, l, pl.ds(0, _L)]
                    a1 = a1 + rowsA[b, s, l, pl.ds(_L, _L)]
                return (a0, a1)

            def bodyB(k, acc):
                a0, a1 = acc
                for u in range(_UNROLL):
                    l = k * _UNROLL + u
                    a0 = a0 + rowsB[b, s, l, pl.ds(0, _L)]
                    a1 = a1 + rowsB[b, s, l, pl.ds(_L, _L)]
                return (a0, a1)

            z = jnp.zeros((_L,), jnp.float32)
            acc = lax.fori_loop(0, _LA // _UNROLL, bodyA, (z, z))
            a0, a1 = lax.fori_loop(0, _LB // _UNROLL, bodyB, acc)
            o = ci * _C + s
            out_v[o, pl.ds(0, _L)] = a0
            out_v[o, pl.ds(_L, _L)] = a1

    fire(0, 0)

    def body(i, carry):
        cc = 2 * i
        fire(cc + 1, 1)
        drain(0)
        accumulate(cc, 0)

        @pl.when(i + 1 < _NCHUNK // 2)
        def _():
            fire(cc + 2, 0)

        drain(1)
        accumulate(cc + 1, 1)
        return carry

    lax.fori_loop(0, _NCHUNK // 2, body, 0)
    pltpu.sync_copy(out_v, out_hbm.at[pl.ds(base, _SPW), :])


_pool = functools.partial(
    pl.kernel,
    out_type=jax.ShapeDtypeStruct((_B, _D), jnp.float32),
    mesh=plsc.VectorSubcoreMesh(core_axis_name="c", subcore_axis_name="s"),
    compiler_params=pltpu.CompilerParams(use_tc_tiling_on_sc=False),
    scratch_types=[
        pltpu.VMEM((2, _C, _H), jnp.int32),
        pltpu.VMEM((2, _C, _LA, _D), jnp.float32),
        pltpu.VMEM((2, _C, _LB, _D), jnp.float32),
        pltpu.VMEM((_SPW, _D), jnp.float32),
        pltpu.SemaphoreType.DMA,
        pltpu.SemaphoreType.DMA,
    ],
)(_pool_body)


def _fc_body(p_ref, w_ref, o_ref):
    o_ref[...] = lax.dot_general(
        p_ref[...], w_ref[...],
        dimension_numbers=(((1,), (1,)), ((), ())),
        preferred_element_type=jnp.float32,
    ) * (1.0 / _H)


_fc = pl.pallas_call(
    _fc_body,
    out_shape=jax.ShapeDtypeStruct((_B, _OUT), jnp.float32),
)


def kernel(x, emb_table, fc_w):
    pooled = _pool(x, emb_table)
    return _fc(pooled, fc_w)


# speedup vs baseline: 2.5505x; 1.0400x over previous
import functools

import jax
import jax.numpy as jnp
from jax import lax
from jax.experimental import pallas as pl
from jax.experimental.pallas import tpu as pltpu
from jax.experimental.pallas import tpu_sc as plsc

_D = 32
_OUT = 10
_B = 16384
_H = 200

_NC = 2
_NS = 16
_NW = _NC * _NS
_SPW = _B // _NW
_C = 8
_NCHUNK = _SPW // _C
_LA = 104
_LB = 96
_L = 16
_UNROLL = 8


def _pool_body(x_hbm, table_hbm, out_hbm, idx_v, rowsA, rowsB, out_v,
               sem0, sem1):
    wid = lax.axis_index("s") * _NC + lax.axis_index("c")
    base = wid * _SPW
    sems = (sem0, sem1)

    def fire(ci, b):
        r0 = base + ci * _C
        pltpu.sync_copy(x_hbm.at[pl.ds(r0, _C), :], idx_v.at[b])
        for s in range(_C):
            pltpu.async_copy(
                table_hbm.at[idx_v.at[b, s, pl.ds(0, _LA)]],
                rowsA.at[b, s], sems[b])
            pltpu.async_copy(
                table_hbm.at[idx_v.at[b, s, pl.ds(_LA, _LB)]],
                rowsB.at[b, s], sems[b])

    def drain(b):
        for s in range(_C):
            pltpu.make_async_copy(
                table_hbm.at[idx_v.at[b, s, pl.ds(0, _LA)]],
                rowsA.at[b, s], sems[b]).wait()
            pltpu.make_async_copy(
                table_hbm.at[idx_v.at[b, s, pl.ds(_LA, _LB)]],
                rowsB.at[b, s], sems[b]).wait()

    def accumulate(ci, b):
        for s in range(_C):
            def bodyA(k, acc):
                a0, a1 = acc
                for u in range(_UNROLL):
                    l = k * _UNROLL + u
                    a0 = a0 + rowsA[b, s, l, pl.ds(0, _L)]
                    a1 = a1 + rowsA[b, s, l, pl.ds(_L, _L)]
                return (a0, a1)

            def bodyB(k, acc):
                a0, a1 = acc
                for u in range(_UNROLL):
                    l = k * _UNROLL + u
                    a0 = a0 + rowsB[b, s, l, pl.ds(0, _L)]
                    a1 = a1 + rowsB[b, s, l, pl.ds(_L, _L)]
                return (a0, a1)

            z = jnp.zeros((_L,), jnp.float32)
            acc = lax.fori_loop(0, _LA // _UNROLL, bodyA, (z, z))
            a0, a1 = lax.fori_loop(0, _LB // _UNROLL, bodyB, acc)
            o = ci * _C + s
            out_v[o, pl.ds(0, _L)] = a0
            out_v[o, pl.ds(_L, _L)] = a1

    fire(0, 0)

    def body(i, carry):
        cc = 2 * i
        fire(cc + 1, 1)
        drain(0)
        accumulate(cc, 0)

        @pl.when(i + 1 < _NCHUNK // 2)
        def _():
            fire(cc + 2, 0)

        drain(1)
        accumulate(cc + 1, 1)
        return carry

    lax.fori_loop(0, _NCHUNK // 2, body, 0)
    pltpu.sync_copy(out_v, out_hbm.at[pl.ds(base, _SPW), :])


_pool = functools.partial(
    pl.kernel,
    out_type=jax.ShapeDtypeStruct((_B, _D), jnp.float32),
    mesh=plsc.VectorSubcoreMesh(core_axis_name="c", subcore_axis_name="s"),
    compiler_params=pltpu.CompilerParams(use_tc_tiling_on_sc=False),
    scratch_types=[
        pltpu.VMEM((2, _C, _H), jnp.int32),
        pltpu.VMEM((2, _C, _LA, _D), jnp.float32),
        pltpu.VMEM((2, _C, _LB, _D), jnp.float32),
        pltpu.VMEM((_SPW, _D), jnp.float32),
        pltpu.SemaphoreType.DMA,
        pltpu.SemaphoreType.DMA,
    ],
)(_pool_body)


def _fc_body(p_ref, w_ref, o_ref):
    o_ref[...] = lax.dot_general(
        p_ref[...], w_ref[...],
        dimension_numbers=(((1,), (1,)), ((), ())),
        preferred_element_type=jnp.float32,
    ) * (1.0 / _H)


_fc = pl.pallas_call(
    _fc_body,
    out_shape=jax.ShapeDtypeStruct((_B, _OUT), jnp.float32),
)


def kernel(x, emb_table, fc_w):
    pooled = _pool(x, emb_table)
    return _fc(pooled, fc_w)
